# baseline (speedup 1.0000x reference)
import jax
import jax.numpy as jnp
from jax.experimental import pallas as pl
from jax.experimental.pallas import tpu as pltpu


def kernel(x, idx):
    B, S, D = x.shape
    R, C = 8, D // 8
    x4 = x.reshape(B, S, R, C)

    def body(idx_ref, x_ref, o_ref):
        o_ref[...] = x_ref[0]

    grid_spec = pltpu.PrefetchScalarGridSpec(
        num_scalar_prefetch=1,
        grid=(B,),
        in_specs=[pl.BlockSpec((1, 1, R, C), lambda b, idx_ref: (b, idx_ref[b], 0, 0))],
        out_specs=pl.BlockSpec((1, R, C), lambda b, idx_ref: (b, 0, 0)),
    )
    out = pl.pallas_call(
        body,
        grid_spec=grid_spec,
        out_shape=jax.ShapeDtypeStruct((B, R, C), x.dtype),
    )(idx.astype(jnp.int32), x4)
    return out.reshape(B, D)


# speedup vs baseline: 51.2642x; 51.2642x over previous
import jax
import jax.numpy as jnp
from jax.experimental import pallas as pl
from jax.experimental.pallas import tpu as pltpu


def kernel(x, idx):
    B, S, D = x.shape
    x_flat = x.reshape(B * S, D)
    gidx = idx.astype(jnp.int32) + jnp.arange(B, dtype=jnp.int32) * S

    def body(idx_ref, x_hbm, o_hbm, sem):
        copies = []
        for b in range(B):
            copies.append(
                pltpu.make_async_copy(
                    x_hbm.at[pl.ds(idx_ref[b], 1)], o_hbm.at[pl.ds(b, 1)], sem
                )
            )
        for c in copies:
            c.start()
        for c in copies:
            c.wait()

    return pl.pallas_call(
        body,
        in_specs=[
            pl.BlockSpec(memory_space=pltpu.SMEM),
            pl.BlockSpec(memory_space=pl.ANY),
        ],
        out_specs=pl.BlockSpec(memory_space=pl.ANY),
        out_shape=jax.ShapeDtypeStruct((B, D), x.dtype),
        scratch_shapes=[pltpu.SemaphoreType.DMA],
    )(gidx, x_flat)
